# baseline (speedup 1.0000x reference)
import functools

import jax
import jax.numpy as jnp
from jax import lax
from jax.experimental import pallas as pl
from jax.experimental.pallas import tpu as pltpu
from jax.experimental.pallas import tpu_sc as plsc

N = 10000
E = 320000
D = 128
HID = 16
G4 = 4 * HID
PW = 80

NC = 2
NS = 16
NW = NC * NS
EPW = E // NW
CH = 80
NCHW = EPW // CH
WR = 80
NRCH = N // WR
RCH_PER_SUB = 8

BLK = 2000


_GATES = ("i", "f", "c", "o")


def _proj_body(x_ref, h_ref, *refs):
    pre_ref, dense_ref = refs[-2], refs[-1]
    prefs = refs[:-2]
    bx, cx, rx, bh, ch, rh, bias = (list(prefs[k::7]) for k in range(7))
    xv = x_ref[...]
    hv = h_ref[...]
    wx = jnp.concatenate([b[0] * c[0, 0] for b, c in zip(bx, cx)], axis=1)
    wh = jnp.concatenate([b[0] * c[0, 0] for b, c in zip(bh, ch)], axis=1)
    pre = jnp.dot(xv, wx, preferred_element_type=jnp.float32)
    pre += jnp.dot(hv, wh, preferred_element_type=jnp.float32)
    pre_ref[...] = pre
    rxc = jnp.concatenate([r[...] for r in rx], axis=1)
    rhc = jnp.concatenate([r[...] for r in rh], axis=1)
    dense = jnp.dot(xv, rxc, preferred_element_type=jnp.float32)
    dense += jnp.dot(hv, rhc, preferred_element_type=jnp.float32)
    dense_ref[...] = dense + jnp.concatenate([b[...] for b in bias], axis=1)


def _proj(x, h, params):
    grid = (N // BLK,)
    full = lambda shape: pl.BlockSpec(shape, lambda i: tuple(0 for _ in shape))
    pargs, pspecs = [], []
    for g in _GATES:
        pargs += [
            params["basis_x_" + g],
            params["comp_x_" + g],
            params["root_x_" + g],
            params["basis_h_" + g],
            params["comp_h_" + g],
            params["root_h_" + g],
            (params["bias_x_" + g] + params["bias_h_" + g]).reshape(1, HID),
        ]
        pspecs += [full((1, D, HID)), full((1, 1)), full((D, HID)),
                   full((1, HID, HID)), full((1, 1)), full((HID, HID)),
                   full((1, HID))]
    return pl.pallas_call(
        _proj_body,
        grid=grid,
        in_specs=[
            pl.BlockSpec((BLK, D), lambda i: (i, 0)),
            pl.BlockSpec((BLK, HID), lambda i: (i, 0)),
        ] + pspecs,
        out_specs=[
            pl.BlockSpec((BLK, G4), lambda i: (i, 0)),
            pl.BlockSpec((BLK, G4), lambda i: (i, 0)),
        ],
        out_shape=[
            jax.ShapeDtypeStruct((N, G4), jnp.float32),
            jax.ShapeDtypeStruct((N, G4), jnp.float32),
        ],
    )(x, h, *pargs)


NBUF = 5


def _seg_body(pre_hbm, edges_hbm, out0, out1, cn0, cn1,
              src_v, dst1_v, dst_v, rows_v, ones_v, stage_v, stage16_v,
              acc_sh, cnt_sh, *sems):
    sg = sems[:NBUF]
    ss = sems[NBUF:]
    cid = lax.axis_index("c")
    sid = lax.axis_index("s")
    wid = cid * NS + sid

    pltpu.sync_copy(edges_hbm.at[0, pl.ds(wid * EPW, EPW)], src_v)
    pltpu.sync_copy(edges_hbm.at[1, pl.ds(wid * EPW, EPW)], dst1_v)

    def _dcopy(k, _):
        row = k // (CH // 16)
        col = (k % (CH // 16)) * 16
        dst_v[row, pl.ds(col, 16)] = dst1_v[pl.ds(k * 16, 16)]
        return _
    lax.fori_loop(0, EPW // 16, _dcopy, None)

    e0 = jnp.where(lax.iota(jnp.int32, 16) == 0, 1.0, 0.0).astype(jnp.float32)

    def _zrow(i, _):
        for j in range(G4 // 16):
            stage_v[i, pl.ds(j * 16, 16)] = jnp.zeros((16,), jnp.float32)
        stage16_v[i, :] = jnp.zeros((16,), jnp.float32)
        ones_v[i, :] = e0
        return _
    lax.fori_loop(0, WR, _zrow, None)
    nch = jnp.minimum(RCH_PER_SUB, NRCH - sid * RCH_PER_SUB)

    def _zchunk(k, _):
        r = (sid * RCH_PER_SUB + k) * WR
        pltpu.sync_copy(stage_v, acc_sh.at[pl.ds(r, WR)])
        pltpu.sync_copy(stage16_v, cnt_sh.at[pl.ds(r, WR)])
        return _
    lax.fori_loop(0, nch, _zchunk, None)
    plsc.subcore_barrier()

    def _gather(j, b, sem):
        pltpu.async_copy(pre_hbm.at[src_v.at[pl.ds(j * CH, CH)]], rows_v.at[b],
                         sem)

    def _gwait(j, b, sem):
        pltpu.make_async_copy(pre_hbm.at[src_v.at[pl.ds(j * CH, CH)]],
                              rows_v.at[b], sem).wait()

    def _scat(j, b, sem):
        pltpu.async_copy(rows_v.at[b], acc_sh.at[dst_v.at[j]], sem, add=True)
        pltpu.async_copy(ones_v, cnt_sh.at[dst_v.at[j]], sem, add=True)

    def _swait(j, b, sem):
        pltpu.make_async_copy(rows_v.at[b], acc_sh.at[dst_v.at[j]], sem).wait()
        pltpu.make_async_copy(ones_v, cnt_sh.at[dst_v.at[j]], sem).wait()

    T = NCHW // NBUF
    for b in range(NBUF):
        _gather(b, b, sg[b])

    def _round(t, _):
        j = t * NBUF
        for b in range(NBUF):
            _gwait(j + b, b, sg[b])
            _scat(j + b, b, ss[b])
        for b in range(NBUF):
            _swait(j + b, b, ss[b])
            @pl.when(t < T - 1)
            def _():
                _gather(j + b + NBUF, b, sg[b])
        return _
    lax.fori_loop(0, T, _round, None)
    plsc.subcore_barrier()

    def _wchunk(k, _):
        r = (sid * RCH_PER_SUB + k) * WR
        @pl.when(cid == 0)
        def _():
            pltpu.sync_copy(acc_sh.at[pl.ds(r, WR)], out0.at[pl.ds(r, WR)])
            pltpu.sync_copy(cnt_sh.at[pl.ds(r, WR)], cn0.at[pl.ds(r, WR)])
        @pl.when(cid == 1)
        def _():
            pltpu.sync_copy(acc_sh.at[pl.ds(r, WR)], out1.at[pl.ds(r, WR)])
            pltpu.sync_copy(cnt_sh.at[pl.ds(r, WR)], cn1.at[pl.ds(r, WR)])
        return _
    lax.fori_loop(0, nch, _wchunk, None)


_seg_sum = functools.partial(
    pl.kernel,
    out_type=[jax.ShapeDtypeStruct((N, G4), jnp.float32)] * 2
    + [jax.ShapeDtypeStruct((N, 16), jnp.float32)] * 2,
    mesh=plsc.VectorSubcoreMesh(core_axis_name="c", subcore_axis_name="s",
                                num_cores=NC, num_subcores=NS),
    compiler_params=pltpu.CompilerParams(use_tc_tiling_on_sc=False),
    scratch_types=[
        pltpu.VMEM((EPW,), jnp.int32),
        pltpu.VMEM((EPW,), jnp.int32),
        pltpu.VMEM((NCHW, CH), jnp.int32),
        pltpu.VMEM((NBUF, CH, G4), jnp.float32),
        pltpu.VMEM((CH, 16), jnp.float32),
        pltpu.VMEM((WR, G4), jnp.float32),
        pltpu.VMEM((WR, 16), jnp.float32),
        pltpu.VMEM_SHARED((N, G4), jnp.float32),
        pltpu.VMEM_SHARED((N, 16), jnp.float32),
    ] + [pltpu.SemaphoreType.DMA] * (2 * NBUF),
)(_seg_body)


def _cell_body(p0_ref, p1_ref, cn0_ref, cn1_ref, dense_ref, c0_ref, lw_ref,
               lb_ref, h_ref, hn_ref, c_ref):
    agg = p0_ref[...] + p1_ref[...]
    cnt = jnp.maximum(cn0_ref[:, 0:1] + cn1_ref[:, 0:1], 1.0)
    z = agg / cnt + dense_ref[...]
    sig = jax.nn.sigmoid(z)
    gi = sig[:, 0:HID]
    gf = sig[:, HID:2 * HID]
    gt = jnp.tanh(z[:, 2 * HID:3 * HID])
    go = sig[:, 3 * HID:4 * HID]
    c_new = gf * c0_ref[...] + gi * gt
    hn = go * jnp.tanh(c_new)
    h_out = jnp.dot(jax.nn.relu(hn), lw_ref[...],
                    preferred_element_type=jnp.float32) + lb_ref[...]
    h_ref[...] = h_out
    hn_ref[...] = hn
    c_ref[...] = c_new


def _cell(p0, p1, cn0, cn1, dense, c0, lw, lb):
    grid = (N // BLK,)
    return pl.pallas_call(
        _cell_body,
        grid=grid,
        in_specs=[
            pl.BlockSpec((BLK, G4), lambda i: (i, 0)),
            pl.BlockSpec((BLK, G4), lambda i: (i, 0)),
            pl.BlockSpec((BLK, 16), lambda i: (i, 0)),
            pl.BlockSpec((BLK, 16), lambda i: (i, 0)),
            pl.BlockSpec((BLK, G4), lambda i: (i, 0)),
            pl.BlockSpec((BLK, HID), lambda i: (i, 0)),
            pl.BlockSpec((HID, 1), lambda i: (0, 0)),
            pl.BlockSpec((1, 1), lambda i: (0, 0)),
        ],
        out_specs=[
            pl.BlockSpec((BLK, 1), lambda i: (i, 0)),
            pl.BlockSpec((BLK, HID), lambda i: (i, 0)),
            pl.BlockSpec((BLK, HID), lambda i: (i, 0)),
        ],
        out_shape=[
            jax.ShapeDtypeStruct((N, 1), jnp.float32),
            jax.ShapeDtypeStruct((N, HID), jnp.float32),
            jax.ShapeDtypeStruct((N, HID), jnp.float32),
        ],
    )(p0, p1, cn0, cn1, dense, c0, lw, lb)


@jax.jit
def _run(x, edge_index, h_0, c_0, params):
    pre, dense = _proj(x, h_0, params)

    p0, p1, cn0, cn1 = _seg_sum(pre, edge_index)

    h, hn, c_new = _cell(p0, p1, cn0, cn1, dense, c_0,
                         params["lin_w"], params["lin_b"][None, :])
    return h, hn, c_new


def kernel(x, edge_index, edge_weight, h_0, c_0, params):
    del edge_weight
    return _run(x, edge_index, h_0, c_0, params)


# speedup vs baseline: 1.0146x; 1.0146x over previous
import functools

import jax
import jax.numpy as jnp
from jax import lax
from jax.experimental import pallas as pl
from jax.experimental.pallas import tpu as pltpu
from jax.experimental.pallas import tpu_sc as plsc

N = 10000
E = 320000
D = 128
HID = 16
G4 = 4 * HID
PW = 80

NC = 2
NS = 16
NW = NC * NS
EPW = E // NW
CH = 80
NCHW = EPW // CH
WR = 80
NRCH = N // WR
RCH_PER_SUB = 8

BLK = 2000


_GATES = ("i", "f", "c", "o")


def _proj_body(x_ref, h_ref, *refs):
    pre_ref, dense_ref = refs[-2], refs[-1]
    prefs = refs[:-2]
    bx, cx, rx, bh, ch, rh, bias = (list(prefs[k::7]) for k in range(7))
    xv = x_ref[...]
    hv = h_ref[...]
    wx = jnp.concatenate([b[0] * c[0, 0] for b, c in zip(bx, cx)], axis=1)
    wh = jnp.concatenate([b[0] * c[0, 0] for b, c in zip(bh, ch)], axis=1)
    pre = jnp.dot(xv, wx, preferred_element_type=jnp.float32)
    pre += jnp.dot(hv, wh, preferred_element_type=jnp.float32)
    ones = jnp.ones((xv.shape[0], 1), jnp.float32)
    pad = jnp.zeros((xv.shape[0], PW - G4 - 1), jnp.float32)
    pre_ref[...] = jnp.concatenate([pre, ones, pad], axis=1)
    rxc = jnp.concatenate([r[...] for r in rx], axis=1)
    rhc = jnp.concatenate([r[...] for r in rh], axis=1)
    dense = jnp.dot(xv, rxc, preferred_element_type=jnp.float32)
    dense += jnp.dot(hv, rhc, preferred_element_type=jnp.float32)
    dense_ref[...] = dense + jnp.concatenate([b[...] for b in bias], axis=1)


def _proj(x, h, params):
    grid = (N // BLK,)
    full = lambda shape: pl.BlockSpec(shape, lambda i: tuple(0 for _ in shape))
    pargs, pspecs = [], []
    for g in _GATES:
        pargs += [
            params["basis_x_" + g],
            params["comp_x_" + g],
            params["root_x_" + g],
            params["basis_h_" + g],
            params["comp_h_" + g],
            params["root_h_" + g],
            (params["bias_x_" + g] + params["bias_h_" + g]).reshape(1, HID),
        ]
        pspecs += [full((1, D, HID)), full((1, 1)), full((D, HID)),
                   full((1, HID, HID)), full((1, 1)), full((HID, HID)),
                   full((1, HID))]
    return pl.pallas_call(
        _proj_body,
        grid=grid,
        in_specs=[
            pl.BlockSpec((BLK, D), lambda i: (i, 0)),
            pl.BlockSpec((BLK, HID), lambda i: (i, 0)),
        ] + pspecs,
        out_specs=[
            pl.BlockSpec((BLK, PW), lambda i: (i, 0)),
            pl.BlockSpec((BLK, G4), lambda i: (i, 0)),
        ],
        out_shape=[
            jax.ShapeDtypeStruct((N, PW), jnp.float32),
            jax.ShapeDtypeStruct((N, G4), jnp.float32),
        ],
    )(x, h, *pargs)


NBUF = 5


def _seg_body(pre_hbm, edges_hbm, out0, out1,
              src_v, dst1_v, dst_v, rows_v, stage_v, acc_sh, *sems):
    sg = sems[:NBUF]
    ss = sems[NBUF:]
    cid = lax.axis_index("c")
    sid = lax.axis_index("s")
    wid = cid * NS + sid

    pltpu.sync_copy(edges_hbm.at[0, pl.ds(wid * EPW, EPW)], src_v)
    pltpu.sync_copy(edges_hbm.at[1, pl.ds(wid * EPW, EPW)], dst1_v)

    def _dcopy(k, _):
        row = k // (CH // 16)
        col = (k % (CH // 16)) * 16
        dst_v[row, pl.ds(col, 16)] = dst1_v[pl.ds(k * 16, 16)]
        return _
    lax.fori_loop(0, EPW // 16, _dcopy, None)

    def _zrow(i, _):
        for j in range(PW // 16):
            stage_v[i, pl.ds(j * 16, 16)] = jnp.zeros((16,), jnp.float32)
        return _
    lax.fori_loop(0, WR, _zrow, None)
    nch = jnp.minimum(RCH_PER_SUB, NRCH - sid * RCH_PER_SUB)

    def _zchunk(k, _):
        r = (sid * RCH_PER_SUB + k) * WR
        pltpu.sync_copy(stage_v, acc_sh.at[pl.ds(r, WR)])
        return _
    lax.fori_loop(0, nch, _zchunk, None)
    plsc.subcore_barrier()

    def _gather(j, b, sem):
        pltpu.async_copy(pre_hbm.at[src_v.at[pl.ds(j * CH, CH)]], rows_v.at[b],
                         sem)

    def _gwait(j, b, sem):
        pltpu.make_async_copy(pre_hbm.at[src_v.at[pl.ds(j * CH, CH)]],
                              rows_v.at[b], sem).wait()

    def _scat(j, b, sem):
        pltpu.async_copy(rows_v.at[b], acc_sh.at[dst_v.at[j]], sem, add=True)

    def _swait(j, b, sem):
        pltpu.make_async_copy(rows_v.at[b], acc_sh.at[dst_v.at[j]], sem).wait()

    T = NCHW // NBUF
    for b in range(NBUF):
        _gather(b, b, sg[b])

    def _round(t, _):
        j = t * NBUF
        for b in range(NBUF):
            _gwait(j + b, b, sg[b])
            _scat(j + b, b, ss[b])
        for b in range(NBUF):
            _swait(j + b, b, ss[b])
            @pl.when(t < T - 1)
            def _():
                _gather(j + b + NBUF, b, sg[b])
        return _
    lax.fori_loop(0, T, _round, None)
    plsc.subcore_barrier()

    def _wchunk(k, _):
        r = (sid * RCH_PER_SUB + k) * WR
        @pl.when(cid == 0)
        def _():
            pltpu.sync_copy(acc_sh.at[pl.ds(r, WR)], out0.at[pl.ds(r, WR)])
        @pl.when(cid == 1)
        def _():
            pltpu.sync_copy(acc_sh.at[pl.ds(r, WR)], out1.at[pl.ds(r, WR)])
        return _
    lax.fori_loop(0, nch, _wchunk, None)


_seg_sum = functools.partial(
    pl.kernel,
    out_type=[jax.ShapeDtypeStruct((N, PW), jnp.float32)] * 2,
    mesh=plsc.VectorSubcoreMesh(core_axis_name="c", subcore_axis_name="s",
                                num_cores=NC, num_subcores=NS),
    compiler_params=pltpu.CompilerParams(use_tc_tiling_on_sc=False),
    scratch_types=[
        pltpu.VMEM((EPW,), jnp.int32),
        pltpu.VMEM((EPW,), jnp.int32),
        pltpu.VMEM((NCHW, CH), jnp.int32),
        pltpu.VMEM((NBUF, CH, PW), jnp.float32),
        pltpu.VMEM((WR, PW), jnp.float32),
        pltpu.VMEM_SHARED((N, PW), jnp.float32),
    ] + [pltpu.SemaphoreType.DMA] * (2 * NBUF),
)(_seg_body)


def _cell_body(p0_ref, p1_ref, dense_ref, c0_ref, lw_ref,
               lb_ref, h_ref, hn_ref, c_ref):
    agg = p0_ref[...] + p1_ref[...]
    cnt = jnp.maximum(agg[:, G4:G4 + 1], 1.0)
    z = agg[:, 0:G4] / cnt + dense_ref[...]
    sig = jax.nn.sigmoid(z)
    gi = sig[:, 0:HID]
    gf = sig[:, HID:2 * HID]
    gt = jnp.tanh(z[:, 2 * HID:3 * HID])
    go = sig[:, 3 * HID:4 * HID]
    c_new = gf * c0_ref[...] + gi * gt
    hn = go * jnp.tanh(c_new)
    h_out = jnp.dot(jax.nn.relu(hn), lw_ref[...],
                    preferred_element_type=jnp.float32) + lb_ref[...]
    h_ref[...] = h_out
    hn_ref[...] = hn
    c_ref[...] = c_new


def _cell(p0, p1, dense, c0, lw, lb):
    grid = (N // BLK,)
    return pl.pallas_call(
        _cell_body,
        grid=grid,
        in_specs=[
            pl.BlockSpec((BLK, PW), lambda i: (i, 0)),
            pl.BlockSpec((BLK, PW), lambda i: (i, 0)),
            pl.BlockSpec((BLK, G4), lambda i: (i, 0)),
            pl.BlockSpec((BLK, HID), lambda i: (i, 0)),
            pl.BlockSpec((HID, 1), lambda i: (0, 0)),
            pl.BlockSpec((1, 1), lambda i: (0, 0)),
        ],
        out_specs=[
            pl.BlockSpec((BLK, 1), lambda i: (i, 0)),
            pl.BlockSpec((BLK, HID), lambda i: (i, 0)),
            pl.BlockSpec((BLK, HID), lambda i: (i, 0)),
        ],
        out_shape=[
            jax.ShapeDtypeStruct((N, 1), jnp.float32),
            jax.ShapeDtypeStruct((N, HID), jnp.float32),
            jax.ShapeDtypeStruct((N, HID), jnp.float32),
        ],
    )(p0, p1, dense, c0, lw, lb)


@jax.jit
def _run(x, edge_index, h_0, c_0, params):
    pre, dense = _proj(x, h_0, params)

    p0, p1 = _seg_sum(pre, edge_index)

    h, hn, c_new = _cell(p0, p1, dense, c_0,
                         params["lin_w"], params["lin_b"][None, :])
    return h, hn, c_new


def kernel(x, edge_index, edge_weight, h_0, c_0, params):
    del edge_weight
    return _run(x, edge_index, h_0, c_0, params)
